# baseline (speedup 1.0000x reference)
import functools
import jax
import jax.numpy as jnp
from jax import lax
from jax.experimental import pallas as pl
from jax.experimental.pallas import tpu as pltpu
from jax.experimental.pallas import tpu_sc as plsc

B, T, D = 4, 4096, 2048
K = T // 2
ROWS, LANES = 32, 128
KROWS = K // LANES


_BT = 1024
_NSTEP = T // _BT


def _scores_kernel(x_ref, w_ref, o_ref):
    o_ref[0] = lax.dot_general(
        w_ref[...], x_ref[...], (((1,), (1,)), ((), ())),
        preferred_element_type=jnp.float32)


def _scores(x2d, w2d, b):
    return pl.pallas_call(
        _scores_kernel,
        grid=(_NSTEP,),
        in_specs=[
            pl.BlockSpec((_BT, D), lambda i, b=b: (b * _NSTEP + i, 0)),
            pl.BlockSpec((1, D), lambda i: (0, 0)),
        ],
        out_specs=pl.BlockSpec((1, 1, _BT), lambda i: (i, 0, 0)),
        out_shape=jax.ShapeDtypeStruct((_NSTEP, 1, _BT), jnp.float32),
    )(x2d, w2d)


def _topk_kernel(b, s_ref, i_ref, f_ref):
    s2 = s_ref[0]
    rows = lax.broadcasted_iota(jnp.int32, (ROWS, LANES), 0)
    lanes = lax.broadcasted_iota(jnp.int32, (ROWS, LANES), 1)
    i2 = rows * LANES + lanes

    def partner(v, d):
        if d < LANES:
            m = (lanes & d) == 0
            return jnp.where(m, pltpu.roll(v, LANES - d, 1),
                             pltpu.roll(v, d, 1)), m
        r = d // LANES
        m = (rows & r) == 0
        return jnp.where(m, pltpu.roll(v, ROWS - r, 0),
                         pltpu.roll(v, r, 0)), m

    kblock = 2
    while kblock <= T:
        d = kblock // 2
        while d >= 1:
            sp, low = partner(s2, d)
            ip, _ = partner(i2, d)
            bfr = (s2 > sp) | ((s2 == sp) & (i2 < ip))
            if kblock < T:
                keep = bfr ^ (~low) ^ (((rows * LANES + lanes) & kblock) != 0)
            else:
                keep = bfr ^ (~low)
            s2 = jnp.where(keep, s2, sp)
            i2 = jnp.where(keep, i2, ip)
            d //= 2
        kblock *= 2

    i_ref[0] = i2[:KROWS]
    f_ref[0] = i2[:KROWS] + b * T


def _topk(scores3, b):
    return pl.pallas_call(
        functools.partial(_topk_kernel, b),
        grid=(1,),
        in_specs=[pl.BlockSpec((1, ROWS, LANES), lambda i: (0, 0, 0))],
        out_specs=[
            pl.BlockSpec((1, KROWS, LANES), lambda i: (0, 0, 0)),
            pl.BlockSpec((1, KROWS, LANES), lambda i: (0, 0, 0)),
        ],
        out_shape=[
            jax.ShapeDtypeStruct((1, KROWS, LANES), jnp.int32),
            jax.ShapeDtypeStruct((1, KROWS, LANES), jnp.int32),
        ],
    )(scores3)


_NC, _NS = 2, 16
_NW = _NC * _NS
_RPW = K // _NW
_CH = 16
_NCHUNK = _RPW // _CH


def _gather_body(b, idx_hbm, x_hbm, out_ref, idx_v, buf0, buf1, gsem):
    wid = lax.axis_index("s") * _NC + lax.axis_index("c")
    base = wid * _NCHUNK
    pltpu.sync_copy(idx_hbm.at[pl.ds(base, _NCHUNK)], idx_v)
    bufs = (buf0, buf1)

    def start_gather(c):
        return pltpu.async_copy(x_hbm.at[idx_v.at[c]], bufs[c % 2], gsem)

    g = [None] * _NCHUNK
    g[0] = start_gather(0)
    out_base = b * K + wid * _RPW
    for c in range(_NCHUNK):
        if c + 1 < _NCHUNK:
            g[c + 1] = start_gather(c + 1)
        g[c].wait()
        pltpu.sync_copy(bufs[c % 2],
                        out_ref.at[pl.ds(out_base + c * _CH, _CH)])


def _gather(idx2d, x2d, out_ref, b):
    mesh = plsc.VectorSubcoreMesh(core_axis_name="c", subcore_axis_name="s")
    f = pl.kernel(
        functools.partial(_gather_body, b),
        out_type=(),
        mesh=mesh,
        scratch_types=[
            pltpu.VMEM((_NCHUNK, _CH), jnp.int32),
            pltpu.VMEM((_CH, D), jnp.float32),
            pltpu.VMEM((_CH, D), jnp.float32),
            pltpu.SemaphoreType.DMA,
        ],
    )
    f(idx2d, x2d, out_ref)


def kernel(x, W):
    x2d = x.reshape(B * T, D)
    w2d = W.reshape(1, D)
    sel_ref = jax.new_ref(lax.empty((B * K, D), jnp.float32))
    score_parts, idx_parts = [], []
    for b in range(B):
        sb = _scores(x2d, w2d, b)
        idx3, flat3 = _topk(sb.reshape(1, ROWS, LANES), b)
        _gather(flat3.reshape(K // _CH, _CH), x2d, sel_ref, b)
        score_parts.append(sb)
        idx_parts.append(idx3)
    scores = jnp.concatenate(score_parts).reshape(B, T)
    indices = jnp.concatenate(idx_parts).reshape(B, K)
    selected = jax.freeze(sel_ref).reshape(B, K, D)
    return (selected, indices, scores)
